# baseline (speedup 1.0000x reference)
import jax
import jax.numpy as jnp
from jax import lax
from jax.experimental import pallas as pl
from jax.experimental.pallas import tpu as pltpu
from jax.experimental.pallas import tpu_sc as plsc

N_WORDS = 1000
B, S = 16, 2048
GROUP_DIMS = (512, 256, 256)
N_CB = 4
OUT_D = sum(GROUP_DIMS)
N_TAB = 12

NC, NS, L = 2, 16, 16
NW = NC * NS
TOK = B * S
T_PER_W = TOK // NW

CHUNK = 32
N_CHUNK = T_PER_W // CHUNK

_HI_MASK = jnp.int32(-65536)


def _wide(v):
  vi = plsc.bitcast(v, jnp.int32)
  e = plsc.bitcast(vi << 16, jnp.float32)
  o = plsc.bitcast(vi & _HI_MASK, jnp.float32)
  return e, o


def _pass_sum(tx, ty, stage, nh, even, odd, add):
  store = plsc.addupdate_scatter if add else plsc.store_scatter

  def body(i, carry):
    for h in range(nh):
      for jj in range(4):
        sl = pl.ds(jj * 32, 32)
        xe, xo = _wide(tx[i, h, sl])
        ye, yo = _wide(ty[i, h, sl])
        row = stage.at[i, h, pl.ds(jj * 32, 32)]
        store(row, [even], xe + ye)
        store(row, [odd], xo + yo)
    return carry

  lax.fori_loop(0, CHUNK, body, 0)


def _sc_body(x5, sos, t00, t01, t02, t03, t10, t11, t12, t13, t20, t21,
             t22, t23, out, idx_v, tA00, tA01, tB00, tB01, g0a, g0b, tA10,
             tA11, tB10, tB11, g1a, g1b, sos_v, sa00, sa01, sb00, sb01,
             sO0a, sO0b, sa10, sa11, sb10, sb11, sO1a, sO1b):
  group_tabs = ((t00, t01, t02, t03), (t10, t11, t12, t13),
                (t20, t21, t22, t23))
  wid = lax.axis_index("s") * NC + lax.axis_index("c")
  b = wid // 2
  b_hi = b // 8
  b_lo = b % 8
  half = wid % 2
  s0 = half * T_PER_W

  even = lax.iota(jnp.int32, L) * 2
  odd = even + 1

  pltpu.sync_copy(x5.at[:, b_hi, pl.ds(half * 8, 8), b_lo, :], idx_v)

  pltpu.sync_copy(sos, sos_v)

  @pl.when(half == 0)
  def _():
    pltpu.sync_copy(sos_v, out.at[0, b_hi, :, b_lo, :])

  def run_group(tabs, nh, h0, jbase, tA, tB, stages, sa, sb, sO):
    def gidx(c, j):
      return idx_v.at[jbase + j, c // 4, pl.ds((c % 4) * CHUNK, CHUNK)]

    def gather(j, c, buf, sem):
      pltpu.async_copy(tabs[j].at[gidx(c, j)], buf, sem)

    def wait_gather(buf, sem):
      pltpu.make_async_copy(tabs[0].at[pl.ds(0, CHUNK)], buf, sem).wait()

    def out_dst(c):
      return out.at[pl.ds(1 + s0 + c * CHUNK, CHUNK), b_hi,
                    pl.ds(h0, nh), b_lo, :]

    def wait_out(p):
      pltpu.make_async_copy(stages[p], out_dst(0), sO[p]).wait()

    def do_chunk(c, p):
      wait_gather(tA[0], sa[0])
      wait_gather(tA[1], sa[1])
      gather(2, c, tB[0], sb[0])
      gather(3, c, tB[1], sb[1])

      @pl.when(c >= 2)
      def _():
        wait_out(p)

      _pass_sum(tA[0], tA[1], stages[p], nh, even, odd, add=False)
      wait_gather(tB[0], sb[0])
      wait_gather(tB[1], sb[1])

      @pl.when(c < N_CHUNK - 1)
      def _():
        gather(0, c + 1, tA[0], sa[0])
        gather(1, c + 1, tA[1], sa[1])

      _pass_sum(tB[0], tB[1], stages[p], nh, even, odd, add=True)
      pltpu.async_copy(stages[p], out_dst(c), sO[p])

    gather(0, 0, tA[0], sa[0])
    gather(1, 0, tA[1], sa[1])

    def pair_body(c2, carry):
      do_chunk(2 * c2, 0)
      do_chunk(2 * c2 + 1, 1)
      return carry

    lax.fori_loop(0, N_CHUNK // 2, pair_body, 0)
    return wait_out

  w0 = run_group(group_tabs[0], 4, 0, 0, (tA00, tA01), (tB00, tB01),
                 (g0a, g0b), (sa00, sa01), (sb00, sb01), (sO0a, sO0b))
  w1 = run_group(group_tabs[1], 2, 4, 4, (tA10, tA11), (tB10, tB11),
                 (g1a, g1b), (sa10, sa11), (sb10, sb11), (sO1a, sO1b))
  w1(0)
  w1(1)
  w2 = run_group(group_tabs[2], 2, 6, 8, (tA10, tA11), (tB10, tB11),
                 (g1a, g1b), (sa10, sa11), (sb10, sb11), (sO1a, sO1b))
  w2(0)
  w2(1)
  w0(0)
  w0(1)


def kernel(x, sos, table_0_0, table_0_1, table_0_2, table_0_3, table_1_0,
           table_1_1, table_1_2, table_1_3, table_2_0, table_2_1, table_2_2,
           table_2_3):
  x5 = (x.transpose((2, 0, 1)).reshape(N_TAB, 2, 8, 16, 128)
        .transpose((0, 1, 3, 2, 4)))
  sos_2d = sos.reshape(8, 128)
  tabs = [
      t.astype(jnp.bfloat16).reshape(N_WORDS + 1, d // 128, 128)
      for t, d in zip(
          (table_0_0, table_0_1, table_0_2, table_0_3, table_1_0, table_1_1,
           table_1_2, table_1_3, table_2_0, table_2_1, table_2_2, table_2_3),
          (512,) * 4 + (256,) * 8)
  ]

  mesh = plsc.VectorSubcoreMesh(
      core_axis_name="c", subcore_axis_name="s", num_cores=NC,
      num_subcores=NS)
  kfn = pl.kernel(
      _sc_body,
      out_type=jax.ShapeDtypeStruct((S + 1, 2, 8, 8, 128), jnp.float32),
      mesh=mesh,
      compiler_params=pltpu.CompilerParams(
          use_tc_tiling_on_sc=False, needs_layout_passes=False),
      scratch_types=[
          pltpu.VMEM((N_TAB, 8, 128), jnp.int32),
          pltpu.VMEM((CHUNK, 4, 128), jnp.bfloat16),
          pltpu.VMEM((CHUNK, 4, 128), jnp.bfloat16),
          pltpu.VMEM((CHUNK, 4, 128), jnp.bfloat16),
          pltpu.VMEM((CHUNK, 4, 128), jnp.bfloat16),
          pltpu.VMEM((CHUNK, 4, 128), jnp.float32),
          pltpu.VMEM((CHUNK, 4, 128), jnp.float32),
          pltpu.VMEM((CHUNK, 2, 128), jnp.bfloat16),
          pltpu.VMEM((CHUNK, 2, 128), jnp.bfloat16),
          pltpu.VMEM((CHUNK, 2, 128), jnp.bfloat16),
          pltpu.VMEM((CHUNK, 2, 128), jnp.bfloat16),
          pltpu.VMEM((CHUNK, 2, 128), jnp.float32),
          pltpu.VMEM((CHUNK, 2, 128), jnp.float32),
          pltpu.VMEM((8, 128), jnp.float32),
      ] + [pltpu.SemaphoreType.DMA] * 12,
  )
  out5 = kfn(x5, sos_2d, *tabs)
  return out5.transpose((1, 3, 0, 2, 4)).reshape(B, S + 1, OUT_D)


# speedup vs baseline: 1.3855x; 1.3855x over previous
import jax
import jax.numpy as jnp
from jax import lax
from jax.experimental import pallas as pl
from jax.experimental.pallas import tpu as pltpu
from jax.experimental.pallas import tpu_sc as plsc

N_WORDS = 1000
B, S = 16, 2048
GROUP_DIMS = (512, 256, 256)
N_CB = 4
OUT_D = sum(GROUP_DIMS)
N_TAB = 12

NC, NS, L = 2, 16, 16
NW = NC * NS
TOK = B * S
T_PER_W = TOK // NW

_HI_MASK = jnp.int32(-65536)


def _wide(v):
  vi = plsc.bitcast(v, jnp.int32)
  e = plsc.bitcast(vi << 16, jnp.float32)
  o = plsc.bitcast(vi & _HI_MASK, jnp.float32)
  return e, o


def _pass4(bufs, stage, chunk, nh, even, odd):
  def body(i, carry):
    for h in range(nh):
      for jj in range(4):
        sl = pl.ds(jj * 32, 32)
        e0, o0 = _wide(bufs[0][i, h, sl])
        e1, o1 = _wide(bufs[1][i, h, sl])
        e2, o2 = _wide(bufs[2][i, h, sl])
        e3, o3 = _wide(bufs[3][i, h, sl])
        row = stage.at[i, h, pl.ds(jj * 32, 32)]
        plsc.store_scatter(row, [even], (e0 + e1) + (e2 + e3))
        plsc.store_scatter(row, [odd], (o0 + o1) + (o2 + o3))
    return carry

  lax.fori_loop(0, chunk, body, 0)


def _sc_body(x5, sos, t00, t01, t02, t03, t10, t11, t12, t13, t20, t21,
             t22, t23, out, idx_v, u00, u01, u02, u03, v00, v01, v02, v03,
             g0a, g0b, u10, u11, u12, u13, v10, v11, v12, v13, g1a, g1b,
             sos_v, su0, sv0, sO0a, sO0b, su1, sv1, sO1a, sO1b):
  group_tabs = ((t00, t01, t02, t03), (t10, t11, t12, t13),
                (t20, t21, t22, t23))
  wid = lax.axis_index("s") * NC + lax.axis_index("c")
  b = wid // 2
  b_hi = b // 8
  b_lo = b % 8
  half = wid % 2
  s0 = half * T_PER_W

  even = lax.iota(jnp.int32, L) * 2
  odd = even + 1

  pltpu.sync_copy(x5.at[:, b_hi, pl.ds(half * 8, 8), b_lo, :], idx_v)

  pltpu.sync_copy(sos, sos_v)

  @pl.when(half == 0)
  def _():
    pltpu.sync_copy(sos_v, out.at[0, b_hi, :, b_lo, :])

  def run_group(tabs, chunk, nh, h0, jbase, setu, setv, stages, su, sv, sO):
    n_chunk = T_PER_W // chunk

    def gidx(c, j):
      off = c * chunk
      return idx_v.at[jbase + j, off // 128, pl.ds(off % 128, chunk)]

    def gather_set(c, bufs, sem):
      for j in range(N_CB):
        pltpu.async_copy(tabs[j].at[gidx(c, j)], bufs[j], sem)

    def wait_set(bufs, sem):
      for j in range(N_CB):
        pltpu.make_async_copy(tabs[0].at[pl.ds(0, chunk)], bufs[j],
                              sem).wait()

    def out_dst(c):
      return out.at[pl.ds(1 + s0 + c * chunk, chunk), b_hi,
                    pl.ds(h0, nh), b_lo, :]

    def wait_out(p):
      pltpu.make_async_copy(stages[p], out_dst(0), sO[p]).wait()

    def do_chunk(c, p, bufs, sem, obufs, osem):
      wait_set(bufs, sem)

      @pl.when(c < n_chunk - 1)
      def _():
        gather_set(c + 1, obufs, osem)

      @pl.when(c >= 2)
      def _():
        wait_out(p)

      _pass4(bufs, stages[p], chunk, nh, even, odd)
      pltpu.async_copy(stages[p], out_dst(c), sO[p])

    gather_set(0, setu, su)

    def pair_body(c2, carry):
      do_chunk(2 * c2, 0, setu, su, setv, sv)
      do_chunk(2 * c2 + 1, 1, setv, sv, setu, su)
      return carry

    lax.fori_loop(0, n_chunk // 2, pair_body, 0)
    return wait_out

  w0 = run_group(group_tabs[0], 16, 4, 0, 0, (u00, u01, u02, u03),
                 (v00, v01, v02, v03), (g0a, g0b), su0, sv0, (sO0a, sO0b))
  w1 = run_group(group_tabs[1], 32, 2, 4, 4, (u10, u11, u12, u13),
                 (v10, v11, v12, v13), (g1a, g1b), su1, sv1, (sO1a, sO1b))
  w1(0)
  w1(1)
  w2 = run_group(group_tabs[2], 32, 2, 6, 8, (u10, u11, u12, u13),
                 (v10, v11, v12, v13), (g1a, g1b), su1, sv1, (sO1a, sO1b))
  w2(0)
  w2(1)
  w0(0)
  w0(1)


def kernel(x, sos, table_0_0, table_0_1, table_0_2, table_0_3, table_1_0,
           table_1_1, table_1_2, table_1_3, table_2_0, table_2_1, table_2_2,
           table_2_3):
  x5 = (x.transpose((2, 0, 1)).reshape(N_TAB, 2, 8, 16, 128)
        .transpose((0, 1, 3, 2, 4)))
  sos_2d = sos.reshape(8, 128)
  tabs = [
      t.astype(jnp.bfloat16).reshape(N_WORDS + 1, d // 128, 128)
      for t, d in zip(
          (table_0_0, table_0_1, table_0_2, table_0_3, table_1_0, table_1_1,
           table_1_2, table_1_3, table_2_0, table_2_1, table_2_2, table_2_3),
          (512,) * 4 + (256,) * 8)
  ]

  mesh = plsc.VectorSubcoreMesh(
      core_axis_name="c", subcore_axis_name="s", num_cores=NC,
      num_subcores=NS)
  kfn = pl.kernel(
      _sc_body,
      out_type=jax.ShapeDtypeStruct((S + 1, 2, 8, 8, 128), jnp.float32),
      mesh=mesh,
      compiler_params=pltpu.CompilerParams(
          use_tc_tiling_on_sc=False, needs_layout_passes=False),
      scratch_types=[
          pltpu.VMEM((N_TAB, 8, 128), jnp.int32),
          pltpu.VMEM((16, 4, 128), jnp.bfloat16),
          pltpu.VMEM((16, 4, 128), jnp.bfloat16),
          pltpu.VMEM((16, 4, 128), jnp.bfloat16),
          pltpu.VMEM((16, 4, 128), jnp.bfloat16),
          pltpu.VMEM((16, 4, 128), jnp.bfloat16),
          pltpu.VMEM((16, 4, 128), jnp.bfloat16),
          pltpu.VMEM((16, 4, 128), jnp.bfloat16),
          pltpu.VMEM((16, 4, 128), jnp.bfloat16),
          pltpu.VMEM((16, 4, 128), jnp.float32),
          pltpu.VMEM((16, 4, 128), jnp.float32),
          pltpu.VMEM((32, 2, 128), jnp.bfloat16),
          pltpu.VMEM((32, 2, 128), jnp.bfloat16),
          pltpu.VMEM((32, 2, 128), jnp.bfloat16),
          pltpu.VMEM((32, 2, 128), jnp.bfloat16),
          pltpu.VMEM((32, 2, 128), jnp.bfloat16),
          pltpu.VMEM((32, 2, 128), jnp.bfloat16),
          pltpu.VMEM((32, 2, 128), jnp.bfloat16),
          pltpu.VMEM((32, 2, 128), jnp.bfloat16),
          pltpu.VMEM((32, 2, 128), jnp.float32),
          pltpu.VMEM((32, 2, 128), jnp.float32),
          pltpu.VMEM((8, 128), jnp.float32),
      ] + [pltpu.SemaphoreType.DMA] * 8,
  )
  out5 = kfn(x5, sos_2d, *tabs)
  return out5.transpose((1, 3, 0, 2, 4)).reshape(B, S + 1, OUT_D)
